# baseline (speedup 1.0000x reference)
import functools

import numpy as np
import jax
import jax.numpy as jnp
from jax import lax
from jax.experimental import pallas as pl
from jax.experimental.pallas import tpu as pltpu
from jax.experimental.pallas import tpu_sc as plsc

N = 10000
E = 160000
W = 16
NC = 2
NS = 16
NW = NC * NS
EPW = E // NW
RPS = N // NS

_f32 = jnp.float32


def _leaky(v):
    return jnp.where(v > 0, v, 0.01 * v)


def _pre_body(x_ref, w1_ref, b1_ref, wl_ref, bl_ref, h_ref, hl_ref):
    h = jnp.dot(x_ref[...], w1_ref[...], preferred_element_type=_f32) + b1_ref[...]
    h_ref[...] = h
    hl_ref[...] = jnp.dot(h, wl_ref[...], preferred_element_type=_f32) + bl_ref[...]


def _pre(x, fc1_W, fc1_b, lin_W, lin_b):
    return pl.pallas_call(
        _pre_body,
        out_shape=[jax.ShapeDtypeStruct((N, W), _f32)] * 2,
    )(x, fc1_W, fc1_b.reshape(1, W), lin_W, lin_b.reshape(1, W))


EB = 2000
GE = E // EB


_bf16 = jnp.bfloat16


def _msg_body(attr_ref, hj_ref, k0w, k0b, k1w, k1b, k2w, k2b, k3w, k3b,
              r_ref, s_ref, msg_ref):
    a = attr_ref[...]
    h1 = _leaky(a * k0w[...] + k0b[...])
    h2 = _leaky(jnp.dot(h1.astype(_bf16), k1w[...],
                        preferred_element_type=_f32) + k1b[...])
    h3 = _leaky(jnp.dot(h2.astype(_bf16), k2w[...],
                        preferred_element_type=_f32) + k2b[...])
    wop = jnp.dot(h3.astype(_bf16), k3w[...],
                  preferred_element_type=_f32) + k3b[...]
    ex = jnp.dot(hj_ref[...].astype(_bf16), r_ref[...],
                 preferred_element_type=_f32)
    msg_ref[...] = jnp.dot((ex * wop).astype(_bf16), s_ref[...],
                           preferred_element_type=_f32)


def _msg(edge_attr, hj, k0_W, k0_b, k1_W, k1_b, k2_W, k2_b, k3_W, k3_b, Rm, Sm):
    def wspec(shape):
        return pl.BlockSpec(shape, lambda i: (0, 0))
    return pl.pallas_call(
        _msg_body,
        grid=(GE,),
        in_specs=[
            pl.BlockSpec((EB, 1), lambda i: (i, 0)),
            pl.BlockSpec((EB, W), lambda i: (i, 0)),
            wspec((1, 32)), wspec((1, 32)),
            wspec((32, 64)), wspec((1, 64)),
            wspec((64, 128)), wspec((1, 128)),
            wspec((128, 256)), wspec((1, 256)),
            wspec((W, 256)), wspec((256, W)),
        ],
        out_specs=pl.BlockSpec((EB, W), lambda i: (i, 0)),
        out_shape=jax.ShapeDtypeStruct((E, W), _f32),
    )(edge_attr, hj, k0_W, k0_b.reshape(1, 32),
      k1_W.astype(_bf16), k1_b.reshape(1, 64),
      k2_W.astype(_bf16), k2_b.reshape(1, 128),
      k3_W.astype(_bf16), k3_b.reshape(1, 256),
      Rm.astype(_bf16), Sm.astype(_bf16))


def _upd1_body(p_ref, c_ref, h_ref, root_ref, kb_ref, wl_ref, bl_ref,
               h1_ref, hl1_ref, cnt_ref):
    cnt = jnp.maximum(c_ref[0] + c_ref[1], 1.0)
    h1 = ((p_ref[0] + p_ref[1]) / cnt
          + jnp.dot(h_ref[...], root_ref[...], preferred_element_type=_f32)
          + kb_ref[...])
    h1_ref[...] = h1
    hl1_ref[...] = jnp.dot(h1, wl_ref[...], preferred_element_type=_f32) + bl_ref[...]
    cnt_ref[...] = cnt


def _upd1(p, c, h, root, kbias, lin_W, lin_b):
    return pl.pallas_call(
        _upd1_body,
        out_shape=[jax.ShapeDtypeStruct((N, W), _f32)] * 3,
    )(p, c, h, root, kbias.reshape(1, W), lin_W, lin_b.reshape(1, W))


def _upd2_body(p_ref, cnt_ref, h_ref, root_ref, kb_ref, wo_ref, bo_ref, out_ref):
    h2 = ((p_ref[0] + p_ref[1]) / cnt_ref[...]
          + jnp.dot(h_ref[...], root_ref[...], preferred_element_type=_f32)
          + kb_ref[...])
    out_ref[...] = jnp.dot(h2, wo_ref[...], preferred_element_type=_f32) + bo_ref[...]


def _upd2(p, cnt, h, root, kbias, fco_W, fco_b):
    return pl.pallas_call(
        _upd2_body,
        out_shape=jax.ShapeDtypeStruct((N, 128), _f32),
    )(p, cnt, h, root, kbias.reshape(1, W), fco_W, fco_b.reshape(1, 128))


def _gather_body(table_hbm, idx_hbm, out_hbm, idx_v, rows_v, sem):
    wid = lax.axis_index("s") * NC + lax.axis_index("c")
    base = wid * EPW
    pltpu.sync_copy(idx_hbm.at[pl.ds(base, EPW)], idx_v)
    pltpu.async_copy(table_hbm.at[idx_v], rows_v, sem).wait()
    pltpu.sync_copy(rows_v, out_hbm.at[pl.ds(base, EPW)])


def _scatter_body(with_count, *refs):
    if with_count:
        (msg_hbm, dst_hbm, ones_hbm, aggr_hbm, cntp_hbm,
         idx_v, rows_v, acc_sh, cnt_sh) = refs
    else:
        (msg_hbm, dst_hbm, aggr_hbm, idx_v, rows_v, acc_sh) = refs
    cid = lax.axis_index("c")
    sid = lax.axis_index("s")
    wid = sid * NC + cid
    base = wid * EPW
    rs = sid * RPS

    def zrow(i, _):
        rows_v[i, :] = jnp.zeros((W,), _f32)
        return 0
    lax.fori_loop(0, RPS, zrow, 0)
    pltpu.sync_copy(rows_v.at[pl.ds(0, RPS)], acc_sh.at[pl.ds(rs, RPS)])
    if with_count:
        pltpu.sync_copy(rows_v.at[pl.ds(0, RPS)], cnt_sh.at[pl.ds(rs, RPS)])
    plsc.subcore_barrier()

    pltpu.sync_copy(dst_hbm.at[pl.ds(base, EPW)], idx_v)
    if with_count:
        pltpu.sync_copy(ones_hbm, rows_v)
        pltpu.sync_copy(rows_v, cnt_sh.at[idx_v], add=True)
    pltpu.sync_copy(msg_hbm.at[pl.ds(base, EPW)], rows_v)
    pltpu.sync_copy(rows_v, acc_sh.at[idx_v], add=True)
    plsc.subcore_barrier()

    pltpu.sync_copy(acc_sh.at[pl.ds(rs, RPS)], aggr_hbm.at[cid, pl.ds(rs, RPS)])
    if with_count:
        pltpu.sync_copy(cnt_sh.at[pl.ds(rs, RPS)], cntp_hbm.at[cid, pl.ds(rs, RPS)])


@functools.lru_cache(maxsize=None)
def _sc_kernels():
    mesh = plsc.VectorSubcoreMesh(core_axis_name="c", subcore_axis_name="s",
                                  num_cores=NC, num_subcores=NS)
    params = pltpu.CompilerParams(use_tc_tiling_on_sc=False)
    gather_k = pl.kernel(
        _gather_body,
        out_type=jax.ShapeDtypeStruct((E, W), _f32),
        mesh=mesh,
        compiler_params=params,
        scratch_types=[
            pltpu.VMEM((EPW,), jnp.int32),
            pltpu.VMEM((EPW, W), _f32),
            pltpu.SemaphoreType.DMA,
        ],
    )
    scatter_cnt_k = pl.kernel(
        functools.partial(_scatter_body, True),
        out_type=(jax.ShapeDtypeStruct((NC, N, W), _f32),
                  jax.ShapeDtypeStruct((NC, N, W), _f32)),
        mesh=mesh,
        compiler_params=params,
        scratch_types=[
            pltpu.VMEM((EPW,), jnp.int32),
            pltpu.VMEM((EPW, W), _f32),
            pltpu.VMEM_SHARED((N, W), _f32),
            pltpu.VMEM_SHARED((N, W), _f32),
        ],
    )
    scatter_k = pl.kernel(
        functools.partial(_scatter_body, False),
        out_type=jax.ShapeDtypeStruct((NC, N, W), _f32),
        mesh=mesh,
        compiler_params=params,
        scratch_types=[
            pltpu.VMEM((EPW,), jnp.int32),
            pltpu.VMEM((EPW, W), _f32),
            pltpu.VMEM_SHARED((N, W), _f32),
        ],
    )
    return gather_k, scatter_cnt_k, scatter_k


_R_const = np.kron(np.eye(W, dtype=np.float32), np.ones((1, W), np.float32))
_S_const = np.tile(np.eye(W, dtype=np.float32), (W, 1))


def kernel(x, edge_index, edge_attr, fc1_W, fc1_b, lin_W, lin_b,
           k0_W, k0_b, k1_W, k1_b, k2_W, k2_b, k3_W, k3_b,
           root, kbias, fco_W, fco_b):
    src = edge_index[0]
    dst = edge_index[1]
    Rm = jnp.asarray(_R_const)
    Sm = jnp.asarray(_S_const)
    ones_rows = jnp.ones((EPW, W), _f32)
    _gather_k, _scatter_cnt_k, _scatter_k = _sc_kernels()

    h0, hl0 = _pre(x, fc1_W, fc1_b, lin_W, lin_b)

    hj1 = _gather_k(hl0, src)
    msg1 = _msg(edge_attr, hj1, k0_W, k0_b, k1_W, k1_b, k2_W, k2_b,
                k3_W, k3_b, Rm, Sm)
    aggr1, cntp = _scatter_cnt_k(msg1, dst, ones_rows)
    h1, hl1, cnt = _upd1(aggr1, cntp, h0, root, kbias, lin_W, lin_b)

    hj2 = _gather_k(hl1, src)
    msg2 = _msg(edge_attr, hj2, k0_W, k0_b, k1_W, k1_b, k2_W, k2_b,
                k3_W, k3_b, Rm, Sm)
    aggr2 = _scatter_k(msg2, dst)
    out = _upd2(aggr2, cnt, h1, root, kbias, fco_W, fco_b)
    return out


# speedup vs baseline: 1.3030x; 1.3030x over previous
import functools

import numpy as np
import jax
import jax.numpy as jnp
from jax import lax
from jax.experimental import pallas as pl
from jax.experimental.pallas import tpu as pltpu
from jax.experimental.pallas import tpu_sc as plsc

N = 10000
E = 160000
W = 16
NC = 2
NS = 16
NW = NC * NS
EPW = E // NW
RPS = N // NS

_f32 = jnp.float32


def _leaky(v):
    return jnp.where(v > 0, v, 0.01 * v)


def _pre_body(x_ref, w1_ref, b1_ref, wl_ref, bl_ref, h_ref, hl_ref):
    h = jnp.dot(x_ref[...], w1_ref[...], preferred_element_type=_f32) + b1_ref[...]
    h_ref[...] = h
    hl_ref[...] = jnp.dot(h, wl_ref[...], preferred_element_type=_f32) + bl_ref[...]


def _pre(x, fc1_W, fc1_b, lin_W, lin_b):
    return pl.pallas_call(
        _pre_body,
        out_shape=[jax.ShapeDtypeStruct((N, W), _f32)] * 2,
    )(x, fc1_W, fc1_b.reshape(1, W), lin_W, lin_b.reshape(1, W))


EB = 3200
GE = E // EB
EBP = EB // 8
EP8 = E // 8


def _msg_body(attr_ref, hjp_ref, k0w, k0b, k1w, k1b, k2w, k2b, k3w, k3b,
              r_ref, s_ref, msgp_ref):
    a = attr_ref[...]
    h1 = _leaky(a * k0w[...] + k0b[...])
    h2 = _leaky(jnp.dot(h1, k1w[...], preferred_element_type=_f32) + k1b[...])
    h3 = _leaky(jnp.dot(h2, k2w[...], preferred_element_type=_f32) + k2b[...])
    wop = jnp.dot(h3, k3w[...], preferred_element_type=_f32) + k3b[...]
    hjp = hjp_ref[...]
    for j in range(8):
        hj_j = hjp[:, 16 * j:16 * (j + 1)]
        ex_j = jnp.dot(hj_j, r_ref[...], preferred_element_type=_f32)
        p_j = ex_j * wop[j * EBP:(j + 1) * EBP, :]
        msgp_ref[:, 16 * j:16 * (j + 1)] = jnp.dot(
            p_j, s_ref[...], preferred_element_type=_f32)


def _msg(attr_t, hjp, k0_W, k0_b, k1_W, k1_b, k2_W, k2_b, k3_W, k3_b, Rm, Sm):
    def wspec(shape):
        return pl.BlockSpec(shape, lambda i: (0, 0))
    return pl.pallas_call(
        _msg_body,
        grid=(GE,),
        in_specs=[
            pl.BlockSpec((EB, 1), lambda i: (i, 0)),
            pl.BlockSpec((EBP, 128), lambda i: (i, 0)),
            wspec((1, 32)), wspec((1, 32)),
            wspec((32, 64)), wspec((1, 64)),
            wspec((64, 128)), wspec((1, 128)),
            wspec((128, 256)), wspec((1, 256)),
            wspec((W, 256)), wspec((256, W)),
        ],
        out_specs=pl.BlockSpec((EBP, 128), lambda i: (i, 0)),
        out_shape=jax.ShapeDtypeStruct((EP8, 128), _f32),
    )(attr_t, hjp, k0_W, k0_b.reshape(1, 32), k1_W, k1_b.reshape(1, 64),
      k2_W, k2_b.reshape(1, 128), k3_W, k3_b.reshape(1, 256), Rm, Sm)


def _upd1_body(p_ref, c_ref, h_ref, root_ref, kb_ref, wl_ref, bl_ref,
               h1_ref, hl1_ref, cnt_ref):
    cnt = jnp.maximum(c_ref[0] + c_ref[1], 1.0)
    h1 = ((p_ref[0] + p_ref[1]) / cnt
          + jnp.dot(h_ref[...], root_ref[...], preferred_element_type=_f32)
          + kb_ref[...])
    h1_ref[...] = h1
    hl1_ref[...] = jnp.dot(h1, wl_ref[...], preferred_element_type=_f32) + bl_ref[...]
    cnt_ref[...] = cnt


def _upd1(p, c, h, root, kbias, lin_W, lin_b):
    return pl.pallas_call(
        _upd1_body,
        out_shape=[jax.ShapeDtypeStruct((N, W), _f32)] * 3,
    )(p, c, h, root, kbias.reshape(1, W), lin_W, lin_b.reshape(1, W))


def _upd2_body(p_ref, cnt_ref, h_ref, root_ref, kb_ref, wo_ref, bo_ref, out_ref):
    h2 = ((p_ref[0] + p_ref[1]) / cnt_ref[...]
          + jnp.dot(h_ref[...], root_ref[...], preferred_element_type=_f32)
          + kb_ref[...])
    out_ref[...] = jnp.dot(h2, wo_ref[...], preferred_element_type=_f32) + bo_ref[...]


def _upd2(p, cnt, h, root, kbias, fco_W, fco_b):
    return pl.pallas_call(
        _upd2_body,
        out_shape=jax.ShapeDtypeStruct((N, 128), _f32),
    )(p, cnt, h, root, kbias.reshape(1, W), fco_W, fco_b.reshape(1, 128))


def _gather_body(table_hbm, idx_hbm, out_hbm, idx_v, rows_v, sem):
    wid = lax.axis_index("s") * NC + lax.axis_index("c")
    base = wid * EPW
    pltpu.sync_copy(idx_hbm.at[pl.ds(base, EPW)], idx_v)
    pltpu.async_copy(table_hbm.at[idx_v], rows_v, sem).wait()
    pltpu.sync_copy(rows_v, out_hbm.at[pl.ds(base, EPW)])


def _scatter_body(with_count, *refs):
    if with_count:
        (msg_hbm, dst_hbm, ones_hbm, aggr_hbm, cntp_hbm,
         idx_v, rows_v, acc_sh, cnt_sh) = refs
    else:
        (msg_hbm, dst_hbm, aggr_hbm, idx_v, rows_v, acc_sh) = refs
    cid = lax.axis_index("c")
    sid = lax.axis_index("s")
    wid = sid * NC + cid
    base = wid * EPW
    rs = sid * RPS

    def zrow(i, _):
        rows_v[i, :] = jnp.zeros((W,), _f32)
        return 0
    lax.fori_loop(0, RPS, zrow, 0)
    pltpu.sync_copy(rows_v.at[pl.ds(0, RPS)], acc_sh.at[pl.ds(rs, RPS)])
    if with_count:
        pltpu.sync_copy(rows_v.at[pl.ds(0, RPS)], cnt_sh.at[pl.ds(rs, RPS)])
    plsc.subcore_barrier()

    pltpu.sync_copy(dst_hbm.at[pl.ds(base, EPW)], idx_v)
    if with_count:
        pltpu.sync_copy(ones_hbm, rows_v)
        pltpu.sync_copy(rows_v, cnt_sh.at[idx_v], add=True)
    pltpu.sync_copy(msg_hbm.at[pl.ds(base, EPW)], rows_v)
    pltpu.sync_copy(rows_v, acc_sh.at[idx_v], add=True)
    plsc.subcore_barrier()

    pltpu.sync_copy(acc_sh.at[pl.ds(rs, RPS)], aggr_hbm.at[cid, pl.ds(rs, RPS)])
    if with_count:
        pltpu.sync_copy(cnt_sh.at[pl.ds(rs, RPS)], cntp_hbm.at[cid, pl.ds(rs, RPS)])


@functools.lru_cache(maxsize=None)
def _sc_kernels():
    mesh = plsc.VectorSubcoreMesh(core_axis_name="c", subcore_axis_name="s",
                                  num_cores=NC, num_subcores=NS)
    params = pltpu.CompilerParams(use_tc_tiling_on_sc=False)
    gather_k = pl.kernel(
        _gather_body,
        out_type=jax.ShapeDtypeStruct((E, W), _f32),
        mesh=mesh,
        compiler_params=params,
        scratch_types=[
            pltpu.VMEM((EPW,), jnp.int32),
            pltpu.VMEM((EPW, W), _f32),
            pltpu.SemaphoreType.DMA,
        ],
    )
    scatter_cnt_k = pl.kernel(
        functools.partial(_scatter_body, True),
        out_type=(jax.ShapeDtypeStruct((NC, N, W), _f32),
                  jax.ShapeDtypeStruct((NC, N, W), _f32)),
        mesh=mesh,
        compiler_params=params,
        scratch_types=[
            pltpu.VMEM((EPW,), jnp.int32),
            pltpu.VMEM((EPW, W), _f32),
            pltpu.VMEM_SHARED((N, W), _f32),
            pltpu.VMEM_SHARED((N, W), _f32),
        ],
    )
    scatter_k = pl.kernel(
        functools.partial(_scatter_body, False),
        out_type=jax.ShapeDtypeStruct((NC, N, W), _f32),
        mesh=mesh,
        compiler_params=params,
        scratch_types=[
            pltpu.VMEM((EPW,), jnp.int32),
            pltpu.VMEM((EPW, W), _f32),
            pltpu.VMEM_SHARED((N, W), _f32),
        ],
    )
    return gather_k, scatter_cnt_k, scatter_k


_R_const = np.kron(np.eye(W, dtype=np.float32), np.ones((1, W), np.float32))
_S_const = np.tile(np.eye(W, dtype=np.float32), (W, 1))


def kernel(x, edge_index, edge_attr, fc1_W, fc1_b, lin_W, lin_b,
           k0_W, k0_b, k1_W, k1_b, k2_W, k2_b, k3_W, k3_b,
           root, kbias, fco_W, fco_b):
    src_s = edge_index[0].reshape(8, EP8).T.reshape(E)
    dst_s = edge_index[1].reshape(8, EP8).T.reshape(E)
    attr_t = edge_attr.reshape(8, GE, EBP).transpose(1, 0, 2).reshape(E, 1)
    Rm = jnp.asarray(_R_const)
    Sm = jnp.asarray(_S_const)
    ones_rows = jnp.ones((EPW, W), _f32)
    _gather_k, _scatter_cnt_k, _scatter_k = _sc_kernels()

    h0, hl0 = _pre(x, fc1_W, fc1_b, lin_W, lin_b)

    hj1 = _gather_k(hl0, src_s).reshape(EP8, 128)
    msg1 = _msg(attr_t, hj1, k0_W, k0_b, k1_W, k1_b, k2_W, k2_b,
                k3_W, k3_b, Rm, Sm).reshape(E, W)
    aggr1, cntp = _scatter_cnt_k(msg1, dst_s, ones_rows)
    h1, hl1, cnt = _upd1(aggr1, cntp, h0, root, kbias, lin_W, lin_b)

    hj2 = _gather_k(hl1, src_s).reshape(EP8, 128)
    msg2 = _msg(attr_t, hj2, k0_W, k0_b, k1_W, k1_b, k2_W, k2_b,
                k3_W, k3_b, Rm, Sm).reshape(E, W)
    aggr2 = _scatter_k(msg2, dst_s)
    out = _upd2(aggr2, cnt, h1, root, kbias, fco_W, fco_b)
    return out
